# Initial kernel scaffold; baseline (speedup 1.0000x reference)
import jax
import jax.numpy as jnp
from jax import lax
from jax.experimental import pallas as pl
from jax.experimental.pallas import tpu as pltpu
from jax.experimental.pallas import tpu_sc as plsc

N_NODES = 100000
DIM = 32
HALF = N_NODES // 2
N_LAYERS = 3
N_EDGES = 1600000

NUM_TILES = 16
BLK = 128
C = 1024
K = C // BLK
CHUNKS = 100
E_T = C * CHUNKS
E_PAD = E_T * NUM_TILES

ROWS_PER_TILE = HALF // NUM_TILES
ZROWS = 125
N_ZCOPY = ROWS_PER_TILE // ZROWS


def _layer_body(t_in, src, dst, w, t_out, acc, srcb, dstb, wb, lidx, rows,
                zbuf, gsem):
    sc = lax.axis_index("c")
    tid = lax.axis_index("s")
    base = sc * HALF
    row0 = tid * ROWS_PER_TILE

    zv = jnp.zeros((16,), jnp.float32)

    def zfill(r, c):
        zbuf[r, pl.ds(0, 16)] = zv
        zbuf[r, pl.ds(16, 16)] = zv
        return c

    lax.fori_loop(0, ZROWS, zfill, 0)

    def zcopy(i, c):
        pltpu.sync_copy(zbuf, acc.at[pl.ds(row0 + i * ZROWS, ZROWS)])
        return c

    lax.fori_loop(0, N_ZCOPY, zcopy, 0)
    plsc.subcore_barrier()

    iota16 = lax.iota(jnp.int32, 16)
    e_base = tid * E_T

    def chunk(ci, carry):
        off = e_base + ci * C
        pltpu.sync_copy(src.at[pl.ds(off, C)], srcb)
        pltpu.sync_copy(dst.at[pl.ds(off, C)], dstb)
        pltpu.sync_copy(w.at[pl.ds(off, C)], wb)
        descs = []
        for j in range(K):
            descs.append(pltpu.async_copy(
                t_in.at[srcb.at[pl.ds(j * BLK, BLK)]],
                rows.at[pl.ds(j * BLK, BLK)], gsem))
        for d in descs:
            d.wait()

        def group(g, c2):
            e0 = g * 16
            dstv = dstb[pl.ds(e0, 16)]
            wv = wb[pl.ds(e0, 16)]
            inr = (dstv >= base) & (dstv < base + HALF)
            wm = jnp.where(inr, wv, 0.0)
            li = jnp.where(inr, dstv - base, 0)
            lidx[g // 8, pl.ds((g % 8) * 16, 16)] = li
            ridx = iota16 + e0
            for dcol in range(DIM):
                cidx = jnp.full((16,), dcol, jnp.int32)
                colv = plsc.load_gather(rows, [ridx, cidx])
                plsc.store_scatter(rows, [ridx, cidx], colv * wm)
            return c2

        lax.fori_loop(0, C // 16, group, 0)

        for j in range(K):
            pltpu.sync_copy(rows.at[pl.ds(j * BLK, BLK)], acc.at[lidx.at[j]],
                            add=True)
        return carry

    lax.fori_loop(0, CHUNKS, chunk, 0)
    plsc.subcore_barrier()

    def wback(i, c):
        r = row0 + i * ZROWS
        pltpu.sync_copy(acc.at[pl.ds(r, ZROWS)], t_out.at[pl.ds(base + r, ZROWS)])
        return c

    lax.fori_loop(0, N_ZCOPY, wback, 0)


_sc_mesh = plsc.VectorSubcoreMesh(core_axis_name="c", subcore_axis_name="s")

_layer = pl.kernel(
    _layer_body,
    out_type=jax.ShapeDtypeStruct((N_NODES, DIM), jnp.float32),
    mesh=_sc_mesh,
    scratch_types=[
        pltpu.VMEM_SHARED((HALF, DIM), jnp.float32),
        pltpu.VMEM((C,), jnp.int32),
        pltpu.VMEM((C,), jnp.int32),
        pltpu.VMEM((C,), jnp.float32),
        pltpu.VMEM((K, BLK), jnp.int32),
        pltpu.VMEM((C, DIM), jnp.float32),
        pltpu.VMEM((ZROWS, DIM), jnp.float32),
        pltpu.SemaphoreType.DMA,
    ],
)


def _mean_body(a, b, c, d, o):
    o[...] = (a[...] + b[...] + c[...] + d[...]) * 0.25


_mean4 = pl.pallas_call(
    _mean_body,
    grid=(100,),
    in_specs=[pl.BlockSpec((1000, DIM), lambda i: (i, 0))] * 4,
    out_specs=pl.BlockSpec((1000, DIM), lambda i: (i, 0)),
    out_shape=jax.ShapeDtypeStruct((N_NODES, DIM), jnp.float32),
)


def kernel(user_emb, item_emb, edge_index, edge_weight):
    all_emb = jnp.concatenate([user_emb, item_emb], axis=0)
    pad = E_PAD - N_EDGES
    zpad_i = jnp.zeros((pad,), jnp.int32)
    src = jnp.concatenate([edge_index[1], zpad_i])
    dst = jnp.concatenate([edge_index[0], zpad_i])
    wp = jnp.concatenate([edge_weight, jnp.zeros((pad,), jnp.float32)])

    t = all_emb
    tables = [all_emb]
    for _ in range(N_LAYERS):
        t = _layer(t, src, dst, wp)
        tables.append(t)
    light_out = _mean4(*tables)
    return (light_out, user_emb, item_emb)


# speedup vs baseline: 5.5260x; 5.5260x over previous
import jax
import jax.numpy as jnp
from jax import lax
from jax.experimental import pallas as pl
from jax.experimental.pallas import tpu as pltpu
from jax.experimental.pallas import tpu_sc as plsc

N_NODES = 100000
DIM = 32
HALF = N_NODES // 2
N_LAYERS = 3
N_EDGES = 1600000

NUM_TILES = 16
BLK = 128
C = 256
K = C // BLK
IBF = 4
CHUNKS = 400
E_T = C * CHUNKS
E_PAD = E_T * NUM_TILES

ZROWS = 125
N_ZCH = HALF // ZROWS // NUM_TILES


def _layer_body(t_in, src, dst, w, t_out, acc, srcb, dstb, wb, lidx, rows,
                isem0, isem1, gsem0, gsem1, ssem0, ssem1):
    sc = lax.axis_index("c")
    tid = lax.axis_index("s")
    base = sc * HALF
    isems, gsems, ssems = (isem0, isem1), (gsem0, gsem1), (ssem0, ssem1)
    e_base = tid * E_T

    def fire_in(gi, b):
        off = e_base + gi * IBF * C
        pltpu.async_copy(src.at[pl.ds(off, IBF * C)], srcb.at[b], isems[b])
        pltpu.async_copy(dst.at[pl.ds(off, IBF * C)], dstb.at[b], isems[b])
        pltpu.async_copy(w.at[pl.ds(off, IBF * C)], wb.at[b], isems[b])

    def wait_in(b):
        for hbm, buf in ((src, srcb), (dst, dstb), (w, wb)):
            pltpu.make_async_copy(hbm.at[pl.ds(0, IBF * C)], buf.at[b],
                                  isems[b]).wait()

    def fire_gather(b, ib, q):
        for j in range(K):
            pltpu.async_copy(
                t_in.at[srcb.at[ib, pl.ds(q * C + j * BLK, BLK)]],
                rows.at[b, pl.ds(j * BLK, BLK)], gsems[b])

    def drain_rows(sems, b):
        for j in range(K):
            pltpu.make_async_copy(t_in.at[pl.ds(0, BLK)],
                                  rows.at[b, pl.ds(j * BLK, BLK)],
                                  sems[b]).wait()

    def fire_scatter(b):
        for j in range(K):
            pltpu.async_copy(rows.at[b, pl.ds(j * BLK, BLK)],
                             acc.at[lidx.at[b * K + j]], ssems[b], add=True)

    def compute(b, ib, q):
        @plsc.parallel_loop(0, C // 16, step=1)
        def group(g):
            e0 = g * 16
            ein = q * C + e0
            dstv = dstb[ib, pl.ds(ein, 16)]
            inr = (dstv >= base) & (dstv < base + HALF)
            li = jnp.where(inr, dstv - base, 0)
            lidx[b * K + g // 8, pl.ds((g % 8) * 16, 16)] = li
            wm = jnp.where(inr, wb[ib, pl.ds(ein, 16)], 0.0)
            for w0 in (0, 8):
                vals = [(rows[b, e0 + w0 + u, pl.ds(0, 16)],
                         rows[b, e0 + w0 + u, pl.ds(16, 16)])
                        for u in range(8)]
                for u in range(8):
                    v0, v1 = vals[u]
                    ws = wm[w0 + u]
                    rows[b, e0 + w0 + u, pl.ds(0, 16)] = v0 * ws
                    rows[b, e0 + w0 + u, pl.ds(16, 16)] = v1 * ws

    fire_in(0, 0)
    fire_in(1, 1)

    zv = jnp.zeros((16,), jnp.float32)

    def zfill(r, c):
        rows[1, r, pl.ds(0, 16)] = zv
        rows[1, r, pl.ds(16, 16)] = zv
        return c

    lax.fori_loop(0, ZROWS, zfill, 0)

    def zcopy(i, c):
        r = (tid + i * NUM_TILES) * ZROWS
        pltpu.sync_copy(rows.at[1, pl.ds(0, ZROWS)], acc.at[pl.ds(r, ZROWS)])
        return c

    lax.fori_loop(0, N_ZCH, zcopy, 0)
    plsc.subcore_barrier()

    def octet(i3, carry):
        for o in range(2 * IBF):
            i = i3 * (2 * IBF) + o
            b, bq = o % 2, 1 - o % 2
            ib, q = (o // IBF) % 2, o % IBF
            if q == 0:
                wait_in(ib)

            @pl.when(i >= 2)
            def _reuse():
                drain_rows(ssems, b)

            fire_gather(b, ib, q)

            qp = (o - 1) % IBF
            iqp = (((o - 1) % (2 * IBF)) // IBF) % 2

            @pl.when(i >= 1)
            def _process():
                drain_rows(gsems, bq)
                compute(bq, iqp, qp)
                fire_scatter(bq)

            if qp == IBF - 1:
                gi = (i - 1) // IBF

                @pl.when((i >= 1) & (gi + 2 < CHUNKS // IBF))
                def _pref():
                    fire_in(gi + 2, iqp)

        return carry

    lax.fori_loop(0, CHUNKS // (2 * IBF), octet, 0)
    drain_rows(gsems, 1)
    compute(1, (((CHUNKS - 1) % (2 * IBF)) // IBF) % 2, (CHUNKS - 1) % IBF)
    fire_scatter(1)
    drain_rows(ssems, 0)
    drain_rows(ssems, 1)
    plsc.subcore_barrier()

    def wback(i, c):
        r = (tid + i * NUM_TILES) * ZROWS
        pltpu.sync_copy(acc.at[pl.ds(r, ZROWS)], t_out.at[pl.ds(base + r, ZROWS)])
        return c

    lax.fori_loop(0, N_ZCH, wback, 0)


_sc_mesh = plsc.VectorSubcoreMesh(core_axis_name="c", subcore_axis_name="s")

_layer = pl.kernel(
    _layer_body,
    out_type=jax.ShapeDtypeStruct((N_NODES, DIM), jnp.float32),
    mesh=_sc_mesh,
    compiler_params=pltpu.CompilerParams(use_tc_tiling_on_sc=False),
    scratch_types=[
        pltpu.VMEM_SHARED((HALF, DIM), jnp.float32),
        pltpu.VMEM((2, IBF * C), jnp.int32),
        pltpu.VMEM((2, IBF * C), jnp.int32),
        pltpu.VMEM((2, IBF * C), jnp.float32),
        pltpu.VMEM((2 * K, BLK), jnp.int32),
        pltpu.VMEM((2, C, DIM), jnp.float32),
        pltpu.SemaphoreType.DMA,
        pltpu.SemaphoreType.DMA,
        pltpu.SemaphoreType.DMA,
        pltpu.SemaphoreType.DMA,
        pltpu.SemaphoreType.DMA,
        pltpu.SemaphoreType.DMA,
    ],
)


MROWS = 250
N_MCH = N_NODES // MROWS
N_WORKERS = 32


def _mean_body(a, b, c, d, o, buf0, buf1):
    wid = lax.axis_index("s") * 2 + lax.axis_index("c")
    n_ch = (N_MCH - wid + N_WORKERS - 1) // N_WORKERS

    def chunk(i, carry):
        r = (wid + i * N_WORKERS) * MROWS
        pltpu.sync_copy(a.at[pl.ds(r, MROWS)], buf0)
        for t in (b, c, d):
            pltpu.sync_copy(t.at[pl.ds(r, MROWS)], buf1)

            def addrow(j, c2):
                buf0[j, pl.ds(0, 16)] = buf0[j, pl.ds(0, 16)] + buf1[j, pl.ds(0, 16)]
                buf0[j, pl.ds(16, 16)] = buf0[j, pl.ds(16, 16)] + buf1[j, pl.ds(16, 16)]
                return c2

            lax.fori_loop(0, MROWS, addrow, 0)

        def scalerow(j, c2):
            buf0[j, pl.ds(0, 16)] = buf0[j, pl.ds(0, 16)] * 0.25
            buf0[j, pl.ds(16, 16)] = buf0[j, pl.ds(16, 16)] * 0.25
            return c2

        lax.fori_loop(0, MROWS, scalerow, 0)
        pltpu.sync_copy(buf0, o.at[pl.ds(r, MROWS)])
        return carry

    lax.fori_loop(0, n_ch, chunk, 0)


_mean4 = pl.kernel(
    _mean_body,
    out_type=jax.ShapeDtypeStruct((N_NODES, DIM), jnp.float32),
    mesh=_sc_mesh,
    compiler_params=pltpu.CompilerParams(use_tc_tiling_on_sc=False),
    scratch_types=[
        pltpu.VMEM((MROWS, DIM), jnp.float32),
        pltpu.VMEM((MROWS, DIM), jnp.float32),
    ],
)


def kernel(user_emb, item_emb, edge_index, edge_weight):
    all_emb = jnp.concatenate([user_emb, item_emb], axis=0)
    pad = E_PAD - N_EDGES
    zpad_i = jnp.zeros((pad,), jnp.int32)
    src = jnp.concatenate([edge_index[1], zpad_i])
    dst = jnp.concatenate([edge_index[0], zpad_i])
    wp = jnp.concatenate([edge_weight, jnp.zeros((pad,), jnp.float32)])

    t = all_emb
    tables = [all_emb]
    for _ in range(N_LAYERS):
        t = _layer(t, src, dst, wp)
        tables.append(t)
    light_out = _mean4(*tables)
    return (light_out, user_emb, item_emb)
